# Initial kernel scaffold; baseline (speedup 1.0000x reference)
import functools

import jax
import jax.numpy as jnp
from jax import lax
from jax.experimental import pallas as pl
from jax.experimental.pallas import tpu as pltpu
from jax.experimental.pallas import tpu_sc as plsc

NC = 2
NS = 16
NW = NC * NS
LANES = 16
CHUNK = 125


def _sc_accumulate(h, src_r, dst_r, compute_deg):
  N, D = h.shape
  _, K, _ = src_r.shape
  rows_per_tile = N // NS

  out_type = [jax.ShapeDtypeStruct((NC, N, D), jnp.float32)]
  if compute_deg:
    out_type.append(jax.ShapeDtypeStruct((NC, N, LANES), jnp.float32))

  scratch = dict(
      acc_sh=pltpu.VMEM_SHARED((N, D), jnp.float32),
      sidx_v=pltpu.VMEM((K, CHUNK), jnp.int32),
      didx_v=pltpu.VMEM((K, CHUNK), jnp.int32),
      rows_v=pltpu.VMEM((CHUNK, D), jnp.float32),
      zero_v=pltpu.VMEM((CHUNK, D), jnp.float32),
      sem=pltpu.SemaphoreType.DMA,
  )
  if compute_deg:
    scratch.update(
        deg_sh=pltpu.VMEM_SHARED((N, LANES), jnp.float32),
        ones_v=pltpu.VMEM((CHUNK, LANES), jnp.float32),
        zdeg_v=pltpu.VMEM((rows_per_tile, LANES), jnp.float32),
    )

  mesh = plsc.VectorSubcoreMesh(core_axis_name="c", subcore_axis_name="s")

  def body(h_hbm, src_hbm, dst_hbm, parts_hbm, *rest):
    if compute_deg:
      deg_hbm = rest[0]
      rest = rest[1:]
    kw = dict(zip(scratch.keys(), rest))
    acc_sh, sidx_v, didx_v = kw["acc_sh"], kw["sidx_v"], kw["didx_v"]
    rows_v, zero_v, sem = kw["rows_v"], kw["zero_v"], kw["sem"]

    c = lax.axis_index("c")
    s = lax.axis_index("s")
    w = c * NS + s

    zf = jnp.zeros((LANES,), jnp.float32)
    vecs_per_row = D // LANES

    def zloop(i, _):
      zero_v[i // vecs_per_row, pl.ds((i % vecs_per_row) * LANES, LANES)] = zf
      return 0
    lax.fori_loop(0, CHUNK * vecs_per_row, zloop, 0)

    if compute_deg:
      ones_v, zdeg_v = kw["ones_v"], kw["zdeg_v"]
      of = jnp.ones((LANES,), jnp.float32)

      def oloop(i, _):
        ones_v[i, pl.ds(0, LANES)] = of
        return 0
      lax.fori_loop(0, CHUNK, oloop, 0)

      def zdloop(i, _):
        zdeg_v[i, pl.ds(0, LANES)] = zf
        return 0
      lax.fori_loop(0, rows_per_tile, zdloop, 0)

    base = s * rows_per_tile
    for t in range(rows_per_tile // CHUNK):
      pltpu.sync_copy(zero_v, acc_sh.at[pl.ds(base + t * CHUNK, CHUNK)])
    if compute_deg:
      pltpu.sync_copy(kw["zdeg_v"], kw["deg_sh"].at[pl.ds(base, rows_per_tile)])
    plsc.subcore_barrier()

    pltpu.sync_copy(src_hbm.at[w], sidx_v)
    pltpu.sync_copy(dst_hbm.at[w], didx_v)

    def chunk_body(j, _):
      pltpu.async_copy(h_hbm.at[sidx_v.at[j]], rows_v, sem).wait()
      pltpu.sync_copy(rows_v, acc_sh.at[didx_v.at[j]], add=True)
      if compute_deg:
        pltpu.sync_copy(kw["ones_v"], kw["deg_sh"].at[didx_v.at[j]], add=True)
      return 0
    lax.fori_loop(0, K, chunk_body, 0)

    plsc.subcore_barrier()

    pltpu.sync_copy(acc_sh.at[pl.ds(base, rows_per_tile)],
                    parts_hbm.at[c, pl.ds(base, rows_per_tile)])
    if compute_deg:
      pltpu.sync_copy(kw["deg_sh"].at[pl.ds(base, rows_per_tile)],
                      deg_hbm.at[c, pl.ds(base, rows_per_tile)])

  kern = pl.kernel(
      body,
      out_type=out_type,
      mesh=mesh,
      scratch_types=list(scratch.values()),
  )
  return kern(h, src_r, dst_r)


def _tc_layer_body(p0, p1, h, d0, d1, w_ref, b_ref, out):
  inv = 1.0 / (d0[:, 0:1] + d1[:, 0:1] + 1.0)
  hn = (p0[...] + p1[...] + h[...]) * inv
  acc = jnp.dot(hn, w_ref[...], preferred_element_type=jnp.float32)
  out[...] = jnp.maximum(acc + b_ref[...], 0.0)


def _tc_layer(parts, h, deg_parts, W, b):
  N, D = h.shape
  BN = 1000
  grid = (N // BN,)
  row_spec = pl.BlockSpec((BN, D), lambda i: (i, 0))
  deg_spec = pl.BlockSpec((BN, LANES), lambda i: (i, 0))
  return pl.pallas_call(
      _tc_layer_body,
      grid=grid,
      in_specs=[row_spec, row_spec, row_spec, deg_spec, deg_spec,
                pl.BlockSpec((D, D), lambda i: (0, 0)),
                pl.BlockSpec((1, D), lambda i: (0, 0))],
      out_specs=row_spec,
      out_shape=jax.ShapeDtypeStruct((N, D), jnp.float32),
  )(parts[0], parts[1], h, deg_parts[0], deg_parts[1], W, b.reshape(1, D))


@jax.jit
def kernel(x, edge_index, W1, b1, W2, b2):
  N, D = x.shape
  E = edge_index.shape[1]
  K = E // (NW * CHUNK)
  src_r = edge_index[0].reshape(NW, K, CHUNK)
  dst_r = edge_index[1].reshape(NW, K, CHUNK)

  parts1, deg_parts = _sc_accumulate(x, src_r, dst_r, compute_deg=True)
  h1 = _tc_layer(parts1, x, deg_parts, W1, b1)
  (parts2,) = _sc_accumulate(h1, src_r, dst_r, compute_deg=False)
  h2 = _tc_layer(parts2, h1, deg_parts, W2, b2)
  return h2


# speedup vs baseline: 6.3538x; 6.3538x over previous
import functools

import jax
import jax.numpy as jnp
from jax import lax
from jax.experimental import pallas as pl
from jax.experimental.pallas import tpu as pltpu
from jax.experimental.pallas import tpu_sc as plsc

NC = 2
NS = 16
NW = NC * NS
LANES = 16
CHUNK = 125
QCH = 16


def _sc_accumulate(h, src_r, dst_r):
  N, D = h.shape
  _, K, _ = src_r.shape
  rpt = N // NS
  nq = K // QCH
  nz = rpt // CHUNK

  scratch = dict(
      acc_sh=pltpu.VMEM_SHARED((N, D), jnp.float32),
      sidx_v=pltpu.VMEM((QCH, CHUNK), jnp.int32),
      didx_v=pltpu.VMEM((QCH, CHUNK), jnp.int32),
      rows_v=pltpu.VMEM((CHUNK, D), jnp.float32),
      sem=pltpu.SemaphoreType.DMA,
  )

  mesh = plsc.VectorSubcoreMesh(core_axis_name="c", subcore_axis_name="s")

  def body(h_hbm, src_hbm, dst_hbm, parts_hbm, acc_sh, sidx_v, didx_v,
           rows_v, sem):
    c = lax.axis_index("c")
    s = lax.axis_index("s")
    w = c * NS + s

    zf = jnp.zeros((LANES,), jnp.float32)
    vecs_per_row = D // LANES

    def zloop(i, _):
      rows_v[i // vecs_per_row, pl.ds((i % vecs_per_row) * LANES, LANES)] = zf
      return 0
    lax.fori_loop(0, CHUNK * vecs_per_row, zloop, 0)

    base = s * rpt
    for t in range(nz):
      pltpu.sync_copy(rows_v, acc_sh.at[pl.ds(base + t * CHUNK, CHUNK)])
    plsc.subcore_barrier()

    def slab_body(q, _):
      pltpu.sync_copy(src_hbm.at[w, pl.ds(q * QCH, QCH)], sidx_v)
      pltpu.sync_copy(dst_hbm.at[w, pl.ds(q * QCH, QCH)], didx_v)

      def chunk_body(j, _):
        pltpu.async_copy(h_hbm.at[sidx_v.at[j]], rows_v, sem).wait()
        pltpu.sync_copy(rows_v, acc_sh.at[didx_v.at[j]], add=True)
        return 0
      lax.fori_loop(0, QCH, chunk_body, 0)
      return 0
    lax.fori_loop(0, nq, slab_body, 0)

    plsc.subcore_barrier()
    pltpu.sync_copy(acc_sh.at[pl.ds(base, rpt)], parts_hbm.at[c, s])

  kern = pl.kernel(
      body,
      out_type=jax.ShapeDtypeStruct((NC, NS, rpt, D), jnp.float32),
      mesh=mesh,
      scratch_types=list(scratch.values()),
  )
  return kern(h, src_r, dst_r)


def _sc_degrees(dst_r, N, D):
  _, K, _ = dst_r.shape
  rpt = N // NS
  nq = K // QCH
  nz = rpt // CHUNK

  scratch = dict(
      deg_sh=pltpu.VMEM_SHARED((N, D), jnp.float32),
      didx_v=pltpu.VMEM((QCH, CHUNK), jnp.int32),
      ones_v=pltpu.VMEM((CHUNK, D), jnp.float32),
  )

  mesh = plsc.VectorSubcoreMesh(core_axis_name="c", subcore_axis_name="s")

  def body(dst_hbm, deg_hbm, deg_sh, didx_v, ones_v):
    c = lax.axis_index("c")
    s = lax.axis_index("s")
    w = c * NS + s

    zf = jnp.zeros((LANES,), jnp.float32)
    vecs_per_row = D // LANES

    def zloop(i, _):
      ones_v[i // vecs_per_row, pl.ds((i % vecs_per_row) * LANES, LANES)] = zf
      return 0
    lax.fori_loop(0, CHUNK * vecs_per_row, zloop, 0)

    base = s * rpt
    for t in range(nz):
      pltpu.sync_copy(ones_v, deg_sh.at[pl.ds(base + t * CHUNK, CHUNK)])
    plsc.subcore_barrier()

    of = jnp.ones((LANES,), jnp.float32)

    def oloop(i, _):
      ones_v[i // vecs_per_row, pl.ds((i % vecs_per_row) * LANES, LANES)] = of
      return 0
    lax.fori_loop(0, CHUNK * vecs_per_row, oloop, 0)

    def slab_body(q, _):
      pltpu.sync_copy(dst_hbm.at[w, pl.ds(q * QCH, QCH)], didx_v)

      def chunk_body(j, _):
        pltpu.sync_copy(ones_v, deg_sh.at[didx_v.at[j]], add=True)
        return 0
      lax.fori_loop(0, QCH, chunk_body, 0)
      return 0
    lax.fori_loop(0, nq, slab_body, 0)

    plsc.subcore_barrier()
    pltpu.sync_copy(deg_sh.at[pl.ds(base, rpt)], deg_hbm.at[c, s])

  kern = pl.kernel(
      body,
      out_type=jax.ShapeDtypeStruct((NC, NS, rpt, D), jnp.float32),
      mesh=mesh,
      scratch_types=list(scratch.values()),
  )
  return kern(dst_r)


def _tc_layer_body(p0, p1, h, d0, d1, w_ref, b_ref, out):
  inv = 1.0 / (d0[:, 0:1] + d1[:, 0:1] + 1.0)
  hn = (p0[...] + p1[...] + h[...]) * inv
  acc = jnp.dot(hn, w_ref[...], preferred_element_type=jnp.float32)
  out[...] = jnp.maximum(acc + b_ref[...], 0.0)


def _tc_layer(parts, h, deg_parts, W, b):
  N, D = h.shape
  BN = 1000
  grid = (N // BN,)
  row_spec = pl.BlockSpec((BN, D), lambda i: (i, 0))
  deg_spec = pl.BlockSpec((BN, D), lambda i: (i, 0))
  return pl.pallas_call(
      _tc_layer_body,
      grid=grid,
      in_specs=[row_spec, row_spec, row_spec, deg_spec, deg_spec,
                pl.BlockSpec((D, D), lambda i: (0, 0)),
                pl.BlockSpec((1, D), lambda i: (0, 0))],
      out_specs=row_spec,
      out_shape=jax.ShapeDtypeStruct((N, D), jnp.float32),
  )(parts[0], parts[1], h, deg_parts[0], deg_parts[1], W, b.reshape(1, D))


@jax.jit
def kernel(x, edge_index, W1, b1, W2, b2):
  N, D = x.shape
  E = edge_index.shape[1]
  K = E // (NW * CHUNK)
  src_r = edge_index[0].reshape(NW, K, CHUNK)
  dst_r = edge_index[1].reshape(NW, K, CHUNK)

  deg_parts = _sc_degrees(dst_r, N, D).reshape(NC, N, D)
  parts1 = _sc_accumulate(x, src_r, dst_r).reshape(NC, N, D)
  h1 = _tc_layer(parts1, x, deg_parts, W1, b1)
  parts2 = _sc_accumulate(h1, src_r, dst_r).reshape(NC, N, D)
  h2 = _tc_layer(parts2, h1, deg_parts, W2, b2)
  return h2
